# baseline (speedup 1.0000x reference)
import functools

import jax
import jax.numpy as jnp
from jax import lax
from jax.experimental import pallas as pl
from jax.experimental.pallas import tpu as pltpu
from jax.experimental.pallas import tpu_sc as plsc


def _edge_mlp_body(leaf_ref, w1_ref, b1_ref, w2_ref, b2_ref, out_ref):
    x = leaf_ref[...]
    h = jnp.dot(x, w1_ref[...], preferred_element_type=jnp.float32) + b1_ref[...]
    h = jnp.maximum(h, 0.0)
    out_ref[...] = (
        jnp.dot(h, w2_ref[...], preferred_element_type=jnp.float32) + b2_ref[...]
    )


def _edge_mlp(leaf, W1, b1, W2, b2, block_e=2048):
    E, F = leaf.shape
    H = W2.shape[1]
    grid = (pl.cdiv(E, block_e),)
    return pl.pallas_call(
        _edge_mlp_body,
        grid=grid,
        in_specs=[
            pl.BlockSpec((block_e, F), lambda i: (i, 0)),
            pl.BlockSpec((F, H), lambda i: (0, 0)),
            pl.BlockSpec((1, H), lambda i: (0, 0)),
            pl.BlockSpec((H, H), lambda i: (0, 0)),
            pl.BlockSpec((1, H), lambda i: (0, 0)),
        ],
        out_specs=pl.BlockSpec((block_e, H), lambda i: (i, 0)),
        out_shape=jax.ShapeDtypeStruct((E, H), jnp.float32),
        compiler_params=pltpu.CompilerParams(
            dimension_semantics=("parallel",),
        ),
    )(leaf, W1, b1.reshape(1, H), W2, b2.reshape(1, H))


_H = 128
_NW = 32
_NPW = 320
_CE = 8000
_GRP = 160
_CAP = 512
_TRIG = _CAP - _GRP


_SH = 18
_LCAP = _CAP + 16
_LROW = 164096


def _make_scan(E):
    n_chunks = E // _CE
    assert n_chunks * _CE == E
    assert E <= (1 << _SH)
    mesh = plsc.VectorSubcoreMesh(core_axis_name="c", subcore_axis_name="s")

    @functools.partial(
        pl.kernel,
        out_type=(
            jax.ShapeDtypeStruct((_NW * _LROW,), jnp.int32),
            jax.ShapeDtypeStruct((_NW * 16,), jnp.int32),
        ),
        mesh=mesh,
        scratch_types=[
            pltpu.VMEM((_CE,), jnp.int32),
            pltpu.VMEM((_LCAP,), jnp.int32),
            pltpu.SMEM((2,), jnp.int32),
        ],
    )
    def scan(idx_hbm, lists_hbm, counts_hbm, idx_v, plist, cnt_ref):
        wid = lax.axis_index("s") * 2 + lax.axis_index("c")
        lo = wid * _NPW
        iota = lax.iota(jnp.int32, 16)
        rot = [(iota + sh) & 15 for sh in (1, 2, 4, 8)]
        sent = jnp.full((16,), _NPW << _SH, jnp.int32)

        cnt_ref[0] = 0
        cnt_ref[1] = 0

        def flush():
            cnt = cnt_ref[0]
            c8 = ((cnt + 7) >> 3) << 3
            plist[pl.ds(cnt, 16)] = sent
            off = pl.multiple_of(cnt_ref[1], 8)
            pltpu.sync_copy(plist.at[pl.ds(0, _CAP)],
                            lists_hbm.at[pl.ds(wid * _LROW + off, _CAP)])
            cnt_ref[1] = off + c8
            cnt_ref[0] = 0

        def chunk_body(c, _):
            pltpu.sync_copy(idx_hbm.at[pl.ds(c * _CE, _CE)], idx_v)
            cbase = c * _CE

            def group_body(g, _):
                for v in range(_GRP // 16):
                    off = g * _GRP + v * 16
                    vec = idx_v[pl.ds(off, 16)]
                    dl = vec - lo
                    m = jnp.logical_and(dl >= 0, dl < _NPW)
                    t = jnp.where(m, 1, 0)
                    for r in rot:
                        t = t + jnp.take(t, r)
                    n = t[0]

                    @pl.when(n > 0)
                    def _(off=off, dl=dl, m=m, n=n):
                        eid = (cbase + off) + iota
                        pk = jnp.where(m, (dl << _SH) + eid, -1)

                        @pl.when(n == 1)
                        def _():
                            t = pk
                            for r in rot:
                                t = jnp.maximum(t, jnp.take(t, r))
                            c0 = cnt_ref[0]
                            plist[pl.ds(c0, 16)] = t
                            cnt_ref[0] = c0 + 1

                        @pl.when(n > 1)
                        def _():
                            for j in range(16):
                                pj = pk[j]

                                @pl.when(pj >= 0)
                                def _(pj=pj):
                                    c0 = cnt_ref[0]
                                    plist[pl.ds(c0, 16)] = jnp.full(
                                        (16,), pj, jnp.int32)
                                    cnt_ref[0] = c0 + 1

                @pl.when(cnt_ref[0] >= _TRIG)
                def _():
                    flush()

                return 0

            lax.fori_loop(0, _CE // _GRP, group_body, 0)
            return 0

        lax.fori_loop(0, n_chunks, chunk_body, 0)

        @pl.when(cnt_ref[0] > 0)
        def _():
            flush()

        plist[pl.ds(0, 16)] = jnp.full((16,), cnt_ref[1], jnp.int32)
        pltpu.sync_copy(plist.at[pl.ds(0, 16)], counts_hbm.at[pl.ds(wid * 16, 16)])

    return scan


def _make_accum(E, NPAD):
    mesh = plsc.VectorSubcoreMesh(core_axis_name="c", subcore_axis_name="s")

    @functools.partial(
        pl.kernel,
        out_type=jax.ShapeDtypeStruct((NPAD, _H), jnp.float32),
        mesh=mesh,
        scratch_types=[
            pltpu.VMEM((_CAP,), jnp.int32),
            pltpu.VMEM((32 * _H,), jnp.float32),
            pltpu.VMEM((_NPW + 1, _H), jnp.float32),
            pltpu.VMEM((16,), jnp.int32),
            pltpu.SemaphoreType.DMA,
            pltpu.SemaphoreType.DMA,
        ],
    )
    def accum(msg_hbm, lists_hbm, counts_hbm, out_hbm,
              plist, rows, acc, cvec, semA, semB):
        wid = lax.axis_index("s") * 2 + lax.axis_index("c")
        lo = wid * _NPW
        iota = lax.iota(jnp.int32, 16)
        neginf = jnp.full((16,), -jnp.inf, jnp.float32)

        def init_a(r, _):
            for q in range(_H // 16):
                acc[r, pl.ds(q * 16, 16)] = neginf
            return 0
        lax.fori_loop(0, _NPW + 1, init_a, 0)

        pltpu.sync_copy(counts_hbm.at[pl.ds(wid * 16, 16)], cvec)
        total = cvec[pl.ds(0, 16)][0]

        def fire(k, base, sem):
            pv = plist[pl.ds(k * 16, 16)]
            ev = jnp.minimum(pv & ((1 << _SH) - 1), E - 1)
            for j in range(16):
                off = pl.multiple_of(ev[j] * _H, 8)
                pltpu.async_copy(
                    msg_hbm.at[pl.ds(off, _H)],
                    rows.at[pl.ds(base + j * _H, _H)], sem)

        def wait16(sem):
            pltpu.make_async_copy(
                msg_hbm.at[pl.ds(0, 16 * _H)],
                rows.at[pl.ds(0, 16 * _H)], sem).wait()

        def accum16(k, base, nrows):
            pv = plist[pl.ds(k * 16, 16)]
            valid = (k * 16 + iota) < nrows
            dloc = jnp.where(valid, pv >> _SH, _NPW)
            for j in range(0, 16, 2):
                d0 = dloc[j]
                d1 = dloc[j + 1]
                r0 = [rows[pl.ds(base + j * _H + q * 16, 16)]
                      for q in range(_H // 16)]
                r1 = [rows[pl.ds(base + (j + 1) * _H + q * 16, 16)]
                      for q in range(_H // 16)]
                a0 = [acc[d0, pl.ds(q * 16, 16)] for q in range(_H // 16)]
                m0 = [jnp.maximum(a, b) for a, b in zip(a0, r0)]
                for q in range(_H // 16):
                    acc[d0, pl.ds(q * 16, 16)] = m0[q]
                a1 = [acc[d1, pl.ds(q * 16, 16)] for q in range(_H // 16)]
                m1 = [jnp.maximum(a, b) for a, b in zip(a1, r1)]
                for q in range(_H // 16):
                    acc[d1, pl.ds(q * 16, 16)] = m1[q]

        def block_body(b, _):
            pltpu.sync_copy(
                lists_hbm.at[pl.ds(wid * _LROW + b * _CAP, _CAP)], plist)
            nrows = jnp.minimum(total - b * _CAP, _CAP)
            kmax = (nrows + 15) // 16
            npairs = (kmax + 1) // 2

            @pl.when(kmax > 0)
            def _():
                fire(0, 0, semA)

            def pair_body(t, _):
                k0 = 2 * t
                k1 = k0 + 1

                @pl.when(k1 < kmax)
                def _():
                    fire(k1, 16 * _H, semB)

                wait16(semA)
                accum16(k0, 0, nrows)

                @pl.when(k1 < kmax)
                def _():
                    @pl.when(k1 + 1 < kmax)
                    def _():
                        fire(k1 + 1, 0, semA)

                    wait16(semB)
                    accum16(k1, 16 * _H, nrows)

                return 0

            lax.fori_loop(0, npairs, pair_body, 0)
            return 0

        nblocks = (total + _CAP - 1) // _CAP
        lax.fori_loop(0, nblocks, block_body, 0)

        def fix_body(r, _):
            for q in range(_H // 16):
                aa = acc[r, pl.ds(q * 16, 16)]
                acc[r, pl.ds(q * 16, 16)] = jnp.where(aa == neginf, 0.0, aa)
            return 0
        lax.fori_loop(0, _NPW, fix_body, 0)
        pltpu.sync_copy(acc.at[pl.ds(0, _NPW)], out_hbm.at[pl.ds(lo, _NPW)])

    return accum


def _node_body(center_ref, agg_ref, gcn_ref, w3_ref, b3_ref, w4_ref, b4_ref,
               out_ref):
    c = center_ref[...]
    a = agg_ref[...]
    H = c.shape[1]
    w3c = w3_ref[0:H, :]
    w3a = w3_ref[H:2 * H, :]
    h = (
        jnp.dot(c, w3c, preferred_element_type=jnp.float32)
        + jnp.dot(a, w3a, preferred_element_type=jnp.float32)
        + b3_ref[...]
    )
    h = jnp.maximum(h, 0.0)
    lang = jnp.dot(h, w4_ref[...], preferred_element_type=jnp.float32) + b4_ref[...]
    g = gcn_ref[...]
    num = jnp.sum(g * lang, axis=1)
    ng = jnp.maximum(jnp.sqrt(jnp.sum(g * g, axis=1)), 1e-8)
    nl = jnp.maximum(jnp.sqrt(jnp.sum(lang * lang, axis=1)), 1e-8)
    out_ref[...] = num / (ng * nl)


def _node_mlp_cosine(center, agg, gcn, W3, b3, W4, b4, block_n=2048):
    N, H = center.shape
    grid = (pl.cdiv(N, block_n),)
    return pl.pallas_call(
        _node_body,
        grid=grid,
        in_specs=[
            pl.BlockSpec((block_n, H), lambda i: (i, 0)),
            pl.BlockSpec((block_n, H), lambda i: (i, 0)),
            pl.BlockSpec((block_n, H), lambda i: (i, 0)),
            pl.BlockSpec((2 * H, H), lambda i: (0, 0)),
            pl.BlockSpec((1, H), lambda i: (0, 0)),
            pl.BlockSpec((H, H), lambda i: (0, 0)),
            pl.BlockSpec((1, H), lambda i: (0, 0)),
        ],
        out_specs=pl.BlockSpec((block_n,), lambda i: (i,)),
        out_shape=jax.ShapeDtypeStruct((N,), jnp.float32),
        compiler_params=pltpu.CompilerParams(
            dimension_semantics=("parallel",),
        ),
    )(center, agg, gcn, W3, b3.reshape(1, H), W4, b4.reshape(1, H))


def kernel(center_node_attr, leaf_node_all, node_idx, gcnfeats,
           W1, b1, W2, b2, W3, b3, W4, b4):
    n = center_node_attr.shape[0]
    E = leaf_node_all.shape[0]
    lists, counts = _make_scan(E)(node_idx.astype(jnp.int32))
    msg = _edge_mlp(leaf_node_all, W1, b1, W2, b2)
    agg_pad = _make_accum(E, _NW * _NPW)(msg.reshape(-1), lists, counts)
    agg = agg_pad[:n]
    return _node_mlp_cosine(center_node_attr, agg, gcnfeats, W3, b3, W4, b4)


# speedup vs baseline: 1.3060x; 1.3060x over previous
import functools

import jax
import jax.numpy as jnp
from jax import lax
from jax.experimental import pallas as pl
from jax.experimental.pallas import tpu as pltpu
from jax.experimental.pallas import tpu_sc as plsc


def _edge_mlp_body(leaf_ref, w1_ref, b1_ref, w2_ref, b2_ref, out_ref):
    x = leaf_ref[...]
    h = jnp.dot(x, w1_ref[...], preferred_element_type=jnp.float32) + b1_ref[...]
    h = jnp.maximum(h, 0.0)
    out_ref[...] = (
        jnp.dot(h, w2_ref[...], preferred_element_type=jnp.float32) + b2_ref[...]
    )


def _edge_mlp(leaf, W1, b1, W2, b2, block_e=2048):
    E, F = leaf.shape
    H = W2.shape[1]
    grid = (pl.cdiv(E, block_e),)
    return pl.pallas_call(
        _edge_mlp_body,
        grid=grid,
        in_specs=[
            pl.BlockSpec((block_e, F), lambda i: (i, 0)),
            pl.BlockSpec((F, H), lambda i: (0, 0)),
            pl.BlockSpec((1, H), lambda i: (0, 0)),
            pl.BlockSpec((H, H), lambda i: (0, 0)),
            pl.BlockSpec((1, H), lambda i: (0, 0)),
        ],
        out_specs=pl.BlockSpec((block_e, H), lambda i: (i, 0)),
        out_shape=jax.ShapeDtypeStruct((E, H), jnp.float32),
        compiler_params=pltpu.CompilerParams(
            dimension_semantics=("parallel",),
        ),
    )(leaf, W1, b1.reshape(1, H), W2, b2.reshape(1, H))


_H = 128
_NW = 32
_NPW = 320
_CE = 8000
_GRP = 160
_CAP = 512
_TRIG = _CAP - _GRP


_SH = 18
_LCAP = _CAP + 16
_LROW = 164096


def _make_scan(E):
    n_chunks = E // _CE
    assert n_chunks * _CE == E
    assert E <= (1 << _SH)
    mesh = plsc.VectorSubcoreMesh(core_axis_name="c", subcore_axis_name="s")

    @functools.partial(
        pl.kernel,
        out_type=(
            jax.ShapeDtypeStruct((_NW * _LROW,), jnp.int32),
            jax.ShapeDtypeStruct((_NW * 16,), jnp.int32),
        ),
        mesh=mesh,
        scratch_types=[
            pltpu.VMEM((_CE,), jnp.int32),
            pltpu.VMEM((_LCAP,), jnp.int32),
            pltpu.SMEM((2,), jnp.int32),
        ],
    )
    def scan(idx_hbm, lists_hbm, counts_hbm, idx_v, plist, cnt_ref):
        wid = lax.axis_index("s") * 2 + lax.axis_index("c")
        lo = wid * _NPW
        iota = lax.iota(jnp.int32, 16)
        rot = [(iota + sh) & 15 for sh in (1, 2, 4, 8)]
        sent = jnp.full((16,), _NPW << _SH, jnp.int32)

        cnt_ref[0] = 0
        cnt_ref[1] = 0

        def flush():
            cnt = cnt_ref[0]
            c8 = ((cnt + 7) >> 3) << 3
            plist[pl.ds(cnt, 16)] = sent
            off = pl.multiple_of(cnt_ref[1], 8)
            pltpu.sync_copy(plist.at[pl.ds(0, _CAP)],
                            lists_hbm.at[pl.ds(wid * _LROW + off, _CAP)])
            cnt_ref[1] = off + c8
            cnt_ref[0] = 0

        def chunk_body(c, _):
            pltpu.sync_copy(idx_hbm.at[pl.ds(c * _CE, _CE)], idx_v)
            cbase = c * _CE

            def group_body(g, _):
                info = []
                for v in range(_GRP // 16):
                    off = g * _GRP + v * 16
                    vec = idx_v[pl.ds(off, 16)]
                    dl = vec - lo
                    m = jnp.logical_and(dl >= 0, dl < _NPW)
                    t = jnp.where(m, 1, 0)
                    for r in rot:
                        t = t + jnp.take(t, r)
                    info.append((off, dl, m, t))
                packs = []
                for h in range(0, _GRP // 16, 5):
                    pk5 = info[h][3]
                    for u in range(1, 5):
                        pk5 = pk5 + (info[h + u][3] << (5 * u))
                    packs.append(pk5[0])
                for v in range(_GRP // 16):
                    off, dl, m, _t = info[v]
                    n = (packs[v // 5] >> (5 * (v % 5))) & 31

                    @pl.when(n > 0)
                    def _(off=off, dl=dl, m=m, n=n):
                        eid = (cbase + off) + iota
                        pk = jnp.where(m, (dl << _SH) + eid, -1)

                        @pl.when(n == 1)
                        def _():
                            t = pk
                            for r in rot:
                                t = jnp.maximum(t, jnp.take(t, r))
                            c0 = cnt_ref[0]
                            plist[pl.ds(c0, 16)] = t
                            cnt_ref[0] = c0 + 1

                        @pl.when(n > 1)
                        def _():
                            for j in range(16):
                                pj = pk[j]

                                @pl.when(pj >= 0)
                                def _(pj=pj):
                                    c0 = cnt_ref[0]
                                    plist[pl.ds(c0, 16)] = jnp.full(
                                        (16,), pj, jnp.int32)
                                    cnt_ref[0] = c0 + 1

                @pl.when(cnt_ref[0] >= _TRIG)
                def _():
                    flush()

                return 0

            lax.fori_loop(0, _CE // _GRP, group_body, 0)
            return 0

        lax.fori_loop(0, n_chunks, chunk_body, 0)

        @pl.when(cnt_ref[0] > 0)
        def _():
            flush()

        plist[pl.ds(0, 16)] = jnp.full((16,), cnt_ref[1], jnp.int32)
        pltpu.sync_copy(plist.at[pl.ds(0, 16)], counts_hbm.at[pl.ds(wid * 16, 16)])

    return scan


def _make_accum(E, NPAD):
    mesh = plsc.VectorSubcoreMesh(core_axis_name="c", subcore_axis_name="s")

    @functools.partial(
        pl.kernel,
        out_type=jax.ShapeDtypeStruct((NPAD, _H), jnp.float32),
        mesh=mesh,
        scratch_types=[
            pltpu.VMEM((_CAP,), jnp.int32),
            pltpu.VMEM((32 * _H,), jnp.float32),
            pltpu.VMEM((_NPW + 1, _H), jnp.float32),
            pltpu.VMEM((16,), jnp.int32),
            pltpu.SemaphoreType.DMA,
            pltpu.SemaphoreType.DMA,
        ],
    )
    def accum(msg_hbm, lists_hbm, counts_hbm, out_hbm,
              plist, rows, acc, cvec, semA, semB):
        wid = lax.axis_index("s") * 2 + lax.axis_index("c")
        lo = wid * _NPW
        iota = lax.iota(jnp.int32, 16)
        neginf = jnp.full((16,), -jnp.inf, jnp.float32)

        def init_a(r, _):
            for q in range(_H // 16):
                acc[r, pl.ds(q * 16, 16)] = neginf
            return 0
        lax.fori_loop(0, _NPW + 1, init_a, 0)

        pltpu.sync_copy(counts_hbm.at[pl.ds(wid * 16, 16)], cvec)
        total = cvec[pl.ds(0, 16)][0]

        def fire(k, base, sem):
            pv = plist[pl.ds(k * 16, 16)]
            ev = jnp.minimum(pv & ((1 << _SH) - 1), E - 1)
            for j in range(16):
                off = pl.multiple_of(ev[j] * _H, 8)
                pltpu.async_copy(
                    msg_hbm.at[pl.ds(off, _H)],
                    rows.at[pl.ds(base + j * _H, _H)], sem)

        def wait16(sem):
            pltpu.make_async_copy(
                msg_hbm.at[pl.ds(0, 16 * _H)],
                rows.at[pl.ds(0, 16 * _H)], sem).wait()

        def accum16(k, base, nrows):
            pv = plist[pl.ds(k * 16, 16)]
            valid = (k * 16 + iota) < nrows
            dloc = jnp.where(valid, pv >> _SH, _NPW)
            for j in range(0, 16, 2):
                d0 = dloc[j]
                d1 = dloc[j + 1]
                r0 = [rows[pl.ds(base + j * _H + q * 16, 16)]
                      for q in range(_H // 16)]
                r1 = [rows[pl.ds(base + (j + 1) * _H + q * 16, 16)]
                      for q in range(_H // 16)]
                a0 = [acc[d0, pl.ds(q * 16, 16)] for q in range(_H // 16)]
                m0 = [jnp.maximum(a, b) for a, b in zip(a0, r0)]
                for q in range(_H // 16):
                    acc[d0, pl.ds(q * 16, 16)] = m0[q]
                a1 = [acc[d1, pl.ds(q * 16, 16)] for q in range(_H // 16)]
                m1 = [jnp.maximum(a, b) for a, b in zip(a1, r1)]
                for q in range(_H // 16):
                    acc[d1, pl.ds(q * 16, 16)] = m1[q]

        def block_body(b, _):
            pltpu.sync_copy(
                lists_hbm.at[pl.ds(wid * _LROW + b * _CAP, _CAP)], plist)
            nrows = jnp.minimum(total - b * _CAP, _CAP)
            kmax = (nrows + 15) // 16
            npairs = (kmax + 1) // 2

            @pl.when(kmax > 0)
            def _():
                fire(0, 0, semA)

            def pair_body(t, _):
                k0 = 2 * t
                k1 = k0 + 1

                @pl.when(k1 < kmax)
                def _():
                    fire(k1, 16 * _H, semB)

                wait16(semA)
                accum16(k0, 0, nrows)

                @pl.when(k1 < kmax)
                def _():
                    @pl.when(k1 + 1 < kmax)
                    def _():
                        fire(k1 + 1, 0, semA)

                    wait16(semB)
                    accum16(k1, 16 * _H, nrows)

                return 0

            lax.fori_loop(0, npairs, pair_body, 0)
            return 0

        nblocks = (total + _CAP - 1) // _CAP
        lax.fori_loop(0, nblocks, block_body, 0)

        def fix_body(r, _):
            for q in range(_H // 16):
                aa = acc[r, pl.ds(q * 16, 16)]
                acc[r, pl.ds(q * 16, 16)] = jnp.where(aa == neginf, 0.0, aa)
            return 0
        lax.fori_loop(0, _NPW, fix_body, 0)
        pltpu.sync_copy(acc.at[pl.ds(0, _NPW)], out_hbm.at[pl.ds(lo, _NPW)])

    return accum


def _node_body(center_ref, agg_ref, gcn_ref, w3_ref, b3_ref, w4_ref, b4_ref,
               out_ref):
    c = center_ref[...]
    a = agg_ref[...]
    H = c.shape[1]
    w3c = w3_ref[0:H, :]
    w3a = w3_ref[H:2 * H, :]
    h = (
        jnp.dot(c, w3c, preferred_element_type=jnp.float32)
        + jnp.dot(a, w3a, preferred_element_type=jnp.float32)
        + b3_ref[...]
    )
    h = jnp.maximum(h, 0.0)
    lang = jnp.dot(h, w4_ref[...], preferred_element_type=jnp.float32) + b4_ref[...]
    g = gcn_ref[...]
    num = jnp.sum(g * lang, axis=1)
    ng = jnp.maximum(jnp.sqrt(jnp.sum(g * g, axis=1)), 1e-8)
    nl = jnp.maximum(jnp.sqrt(jnp.sum(lang * lang, axis=1)), 1e-8)
    out_ref[...] = num / (ng * nl)


def _node_mlp_cosine(center, agg, gcn, W3, b3, W4, b4, block_n=2048):
    N, H = center.shape
    grid = (pl.cdiv(N, block_n),)
    return pl.pallas_call(
        _node_body,
        grid=grid,
        in_specs=[
            pl.BlockSpec((block_n, H), lambda i: (i, 0)),
            pl.BlockSpec((block_n, H), lambda i: (i, 0)),
            pl.BlockSpec((block_n, H), lambda i: (i, 0)),
            pl.BlockSpec((2 * H, H), lambda i: (0, 0)),
            pl.BlockSpec((1, H), lambda i: (0, 0)),
            pl.BlockSpec((H, H), lambda i: (0, 0)),
            pl.BlockSpec((1, H), lambda i: (0, 0)),
        ],
        out_specs=pl.BlockSpec((block_n,), lambda i: (i,)),
        out_shape=jax.ShapeDtypeStruct((N,), jnp.float32),
        compiler_params=pltpu.CompilerParams(
            dimension_semantics=("parallel",),
        ),
    )(center, agg, gcn, W3, b3.reshape(1, H), W4, b4.reshape(1, H))


def kernel(center_node_attr, leaf_node_all, node_idx, gcnfeats,
           W1, b1, W2, b2, W3, b3, W4, b4):
    n = center_node_attr.shape[0]
    E = leaf_node_all.shape[0]
    lists, counts = _make_scan(E)(node_idx.astype(jnp.int32))
    msg = _edge_mlp(leaf_node_all, W1, b1, W2, b2)
    agg_pad = _make_accum(E, _NW * _NPW)(msg.reshape(-1), lists, counts)
    agg = agg_pad[:n]
    return _node_mlp_cosine(center_node_attr, agg, gcnfeats, W3, b3, W4, b4)


# speedup vs baseline: 1.3849x; 1.0604x over previous
import functools

import jax
import jax.numpy as jnp
from jax import lax
from jax.experimental import pallas as pl
from jax.experimental.pallas import tpu as pltpu
from jax.experimental.pallas import tpu_sc as plsc


def _edge_mlp_body(leaf_ref, w1_ref, b1_ref, w2_ref, b2_ref, out_ref):
    x = leaf_ref[...]
    h = jnp.dot(x, w1_ref[...], preferred_element_type=jnp.float32) + b1_ref[...]
    h = jnp.maximum(h, 0.0)
    out_ref[...] = (
        jnp.dot(h, w2_ref[...], preferred_element_type=jnp.float32) + b2_ref[...]
    )


def _edge_mlp(leaf, W1, b1, W2, b2, block_e=2048):
    E, F = leaf.shape
    H = W2.shape[1]
    grid = (pl.cdiv(E, block_e),)
    return pl.pallas_call(
        _edge_mlp_body,
        grid=grid,
        in_specs=[
            pl.BlockSpec((block_e, F), lambda i: (i, 0)),
            pl.BlockSpec((F, H), lambda i: (0, 0)),
            pl.BlockSpec((1, H), lambda i: (0, 0)),
            pl.BlockSpec((H, H), lambda i: (0, 0)),
            pl.BlockSpec((1, H), lambda i: (0, 0)),
        ],
        out_specs=pl.BlockSpec((block_e, H), lambda i: (i, 0)),
        out_shape=jax.ShapeDtypeStruct((E, H), jnp.float32),
        compiler_params=pltpu.CompilerParams(
            dimension_semantics=("parallel",),
        ),
    )(leaf, W1, b1.reshape(1, H), W2, b2.reshape(1, H))


_H = 128
_NW = 32
_NPW = 320
_CE = 8000
_GRP = 160
_CAP = 512
_TRIG = _CAP - _GRP


_SH = 18
_LCAP = _CAP + 16
_LROW = 164096


def _make_scan(E):
    n_chunks = E // _CE
    assert n_chunks * _CE == E
    assert E <= (1 << _SH)
    mesh = plsc.VectorSubcoreMesh(core_axis_name="c", subcore_axis_name="s")

    @functools.partial(
        pl.kernel,
        out_type=(
            jax.ShapeDtypeStruct((_NW * _LROW,), jnp.int32),
            jax.ShapeDtypeStruct((_NW * 16,), jnp.int32),
        ),
        mesh=mesh,
        scratch_types=[
            pltpu.VMEM((_CE,), jnp.int32),
            pltpu.VMEM((_LCAP,), jnp.int32),
            pltpu.SMEM((2,), jnp.int32),
        ],
    )
    def scan(idx_hbm, lists_hbm, counts_hbm, idx_v, plist, cnt_ref):
        wid = lax.axis_index("s") * 2 + lax.axis_index("c")
        lo = wid * _NPW
        iota = lax.iota(jnp.int32, 16)
        rot = [(iota + sh) & 15 for sh in (1, 2, 4, 8)]
        sent = jnp.full((16,), _NPW << _SH, jnp.int32)

        cnt_ref[0] = 0
        cnt_ref[1] = 0

        def flush():
            cnt = cnt_ref[0]
            c8 = ((cnt + 7) >> 3) << 3
            plist[pl.ds(cnt, 16)] = sent
            off = pl.multiple_of(cnt_ref[1], 8)
            pltpu.sync_copy(plist.at[pl.ds(0, _CAP)],
                            lists_hbm.at[pl.ds(wid * _LROW + off, _CAP)])
            cnt_ref[1] = off + c8
            cnt_ref[0] = 0

        def chunk_body(c, _):
            pltpu.sync_copy(idx_hbm.at[pl.ds(c * _CE, _CE)], idx_v)
            cbase = c * _CE

            def group_body(g, _):
                info = []
                for v in range(_GRP // 16):
                    off = g * _GRP + v * 16
                    vec = idx_v[pl.ds(off, 16)]
                    dl = vec - lo
                    m = jnp.logical_and(dl >= 0, dl < _NPW)
                    info.append((off, dl, m))
                packs = []
                for h in range(0, _GRP // 16, 5):
                    t = jnp.where(info[h][2], 1, 0)
                    for u in range(1, 5):
                        t = t + jnp.where(info[h + u][2], 1 << (5 * u), 0)
                    for r in rot:
                        t = t + jnp.take(t, r)
                    packs.append(t[0])
                for v in range(_GRP // 16):
                    off, dl, m = info[v]
                    n = (packs[v // 5] >> (5 * (v % 5))) & 31

                    @pl.when(n > 0)
                    def _(off=off, dl=dl, m=m, n=n):
                        eid = (cbase + off) + iota
                        pk = jnp.where(m, (dl << _SH) + eid, -1)

                        @pl.when(n == 1)
                        def _():
                            t = pk
                            for r in rot:
                                t = jnp.maximum(t, jnp.take(t, r))
                            c0 = cnt_ref[0]
                            plist[pl.ds(c0, 16)] = t
                            cnt_ref[0] = c0 + 1

                        @pl.when(n > 1)
                        def _():
                            for j in range(16):
                                pj = pk[j]

                                @pl.when(pj >= 0)
                                def _(pj=pj):
                                    c0 = cnt_ref[0]
                                    plist[pl.ds(c0, 16)] = jnp.full(
                                        (16,), pj, jnp.int32)
                                    cnt_ref[0] = c0 + 1

                @pl.when(cnt_ref[0] >= _TRIG)
                def _():
                    flush()

                return 0

            lax.fori_loop(0, _CE // _GRP, group_body, 0)
            return 0

        lax.fori_loop(0, n_chunks, chunk_body, 0)

        @pl.when(cnt_ref[0] > 0)
        def _():
            flush()

        plist[pl.ds(0, 16)] = jnp.full((16,), cnt_ref[1], jnp.int32)
        pltpu.sync_copy(plist.at[pl.ds(0, 16)], counts_hbm.at[pl.ds(wid * 16, 16)])

    return scan


def _make_accum(E, NPAD):
    mesh = plsc.VectorSubcoreMesh(core_axis_name="c", subcore_axis_name="s")

    @functools.partial(
        pl.kernel,
        out_type=jax.ShapeDtypeStruct((NPAD, _H), jnp.float32),
        mesh=mesh,
        scratch_types=[
            pltpu.VMEM((_CAP,), jnp.int32),
            pltpu.VMEM((32 * _H,), jnp.float32),
            pltpu.VMEM((_NPW + 1, _H), jnp.float32),
            pltpu.VMEM((16,), jnp.int32),
            pltpu.SemaphoreType.DMA,
            pltpu.SemaphoreType.DMA,
        ],
    )
    def accum(msg_hbm, lists_hbm, counts_hbm, out_hbm,
              plist, rows, acc, cvec, semA, semB):
        wid = lax.axis_index("s") * 2 + lax.axis_index("c")
        lo = wid * _NPW
        iota = lax.iota(jnp.int32, 16)
        neginf = jnp.full((16,), -jnp.inf, jnp.float32)

        def init_a(r, _):
            for q in range(_H // 16):
                acc[r, pl.ds(q * 16, 16)] = neginf
            return 0
        lax.fori_loop(0, _NPW + 1, init_a, 0)

        pltpu.sync_copy(counts_hbm.at[pl.ds(wid * 16, 16)], cvec)
        total = cvec[pl.ds(0, 16)][0]

        def fire(k, base, sem):
            pv = plist[pl.ds(k * 16, 16)]
            ev = jnp.minimum(pv & ((1 << _SH) - 1), E - 1)
            for j in range(16):
                off = pl.multiple_of(ev[j] * _H, 8)
                pltpu.async_copy(
                    msg_hbm.at[pl.ds(off, _H)],
                    rows.at[pl.ds(base + j * _H, _H)], sem)

        def wait16(sem):
            pltpu.make_async_copy(
                msg_hbm.at[pl.ds(0, 16 * _H)],
                rows.at[pl.ds(0, 16 * _H)], sem).wait()

        def accum16(k, base, nrows):
            pv = plist[pl.ds(k * 16, 16)]
            valid = (k * 16 + iota) < nrows
            dloc = jnp.where(valid, pv >> _SH, _NPW)
            for j in range(0, 16, 2):
                d0 = dloc[j]
                d1 = dloc[j + 1]
                r0 = [rows[pl.ds(base + j * _H + q * 16, 16)]
                      for q in range(_H // 16)]
                r1 = [rows[pl.ds(base + (j + 1) * _H + q * 16, 16)]
                      for q in range(_H // 16)]
                a0 = [acc[d0, pl.ds(q * 16, 16)] for q in range(_H // 16)]
                m0 = [jnp.maximum(a, b) for a, b in zip(a0, r0)]
                for q in range(_H // 16):
                    acc[d0, pl.ds(q * 16, 16)] = m0[q]
                a1 = [acc[d1, pl.ds(q * 16, 16)] for q in range(_H // 16)]
                m1 = [jnp.maximum(a, b) for a, b in zip(a1, r1)]
                for q in range(_H // 16):
                    acc[d1, pl.ds(q * 16, 16)] = m1[q]

        def block_body(b, _):
            pltpu.sync_copy(
                lists_hbm.at[pl.ds(wid * _LROW + b * _CAP, _CAP)], plist)
            nrows = jnp.minimum(total - b * _CAP, _CAP)
            kmax = (nrows + 15) // 16
            npairs = (kmax + 1) // 2

            @pl.when(kmax > 0)
            def _():
                fire(0, 0, semA)

            def pair_body(t, _):
                k0 = 2 * t
                k1 = k0 + 1

                @pl.when(k1 < kmax)
                def _():
                    fire(k1, 16 * _H, semB)

                wait16(semA)
                accum16(k0, 0, nrows)

                @pl.when(k1 < kmax)
                def _():
                    @pl.when(k1 + 1 < kmax)
                    def _():
                        fire(k1 + 1, 0, semA)

                    wait16(semB)
                    accum16(k1, 16 * _H, nrows)

                return 0

            lax.fori_loop(0, npairs, pair_body, 0)
            return 0

        nblocks = (total + _CAP - 1) // _CAP
        lax.fori_loop(0, nblocks, block_body, 0)

        def fix_body(r, _):
            for q in range(_H // 16):
                aa = acc[r, pl.ds(q * 16, 16)]
                acc[r, pl.ds(q * 16, 16)] = jnp.where(aa == neginf, 0.0, aa)
            return 0
        lax.fori_loop(0, _NPW, fix_body, 0)
        pltpu.sync_copy(acc.at[pl.ds(0, _NPW)], out_hbm.at[pl.ds(lo, _NPW)])

    return accum


def _node_body(center_ref, agg_ref, gcn_ref, w3_ref, b3_ref, w4_ref, b4_ref,
               out_ref):
    c = center_ref[...]
    a = agg_ref[...]
    H = c.shape[1]
    w3c = w3_ref[0:H, :]
    w3a = w3_ref[H:2 * H, :]
    h = (
        jnp.dot(c, w3c, preferred_element_type=jnp.float32)
        + jnp.dot(a, w3a, preferred_element_type=jnp.float32)
        + b3_ref[...]
    )
    h = jnp.maximum(h, 0.0)
    lang = jnp.dot(h, w4_ref[...], preferred_element_type=jnp.float32) + b4_ref[...]
    g = gcn_ref[...]
    num = jnp.sum(g * lang, axis=1)
    ng = jnp.maximum(jnp.sqrt(jnp.sum(g * g, axis=1)), 1e-8)
    nl = jnp.maximum(jnp.sqrt(jnp.sum(lang * lang, axis=1)), 1e-8)
    out_ref[...] = num / (ng * nl)


def _node_mlp_cosine(center, agg, gcn, W3, b3, W4, b4, block_n=2048):
    N, H = center.shape
    grid = (pl.cdiv(N, block_n),)
    return pl.pallas_call(
        _node_body,
        grid=grid,
        in_specs=[
            pl.BlockSpec((block_n, H), lambda i: (i, 0)),
            pl.BlockSpec((block_n, H), lambda i: (i, 0)),
            pl.BlockSpec((block_n, H), lambda i: (i, 0)),
            pl.BlockSpec((2 * H, H), lambda i: (0, 0)),
            pl.BlockSpec((1, H), lambda i: (0, 0)),
            pl.BlockSpec((H, H), lambda i: (0, 0)),
            pl.BlockSpec((1, H), lambda i: (0, 0)),
        ],
        out_specs=pl.BlockSpec((block_n,), lambda i: (i,)),
        out_shape=jax.ShapeDtypeStruct((N,), jnp.float32),
        compiler_params=pltpu.CompilerParams(
            dimension_semantics=("parallel",),
        ),
    )(center, agg, gcn, W3, b3.reshape(1, H), W4, b4.reshape(1, H))


def kernel(center_node_attr, leaf_node_all, node_idx, gcnfeats,
           W1, b1, W2, b2, W3, b3, W4, b4):
    n = center_node_attr.shape[0]
    E = leaf_node_all.shape[0]
    lists, counts = _make_scan(E)(node_idx.astype(jnp.int32))
    msg = _edge_mlp(leaf_node_all, W1, b1, W2, b2)
    agg_pad = _make_accum(E, _NW * _NPW)(msg.reshape(-1), lists, counts)
    agg = agg_pad[:n]
    return _node_mlp_cosine(center_node_attr, agg, gcnfeats, W3, b3, W4, b4)
